# Initial kernel scaffold; baseline (speedup 1.0000x reference)
import jax
import jax.numpy as jnp
from jax.experimental import pallas as pl


def kernel(features, edge_index, W0, b0, W1, b1, W2, b2):
    raise NotImplementedError("write your pallas kernel here")


# speedup vs baseline: 8.8989x; 8.8989x over previous
import jax
import jax.numpy as jnp
from jax import lax
from jax.experimental import pallas as pl
from jax.experimental.pallas import tpu as pltpu
from jax.experimental.pallas import tpu_sc as plsc

N = 10000
NP = 10240
E = 320000
F_IN = 128
F_HID = 128
F_OUT = 40
FH = 64

NC, NS = 2, 16
NWORK = NC * NS
EPW = E // NWORK
WIN = 100
NWIN = EPW // WIN
RPS = NP // NS

_MESH = plsc.VectorSubcoreMesh(core_axis_name="c", subcore_axis_name="s")


def _degree_body(srcb, dstb, ones_h, zz, out, isrc, idst, ones_v, acc_o, acc_i):
    c = lax.axis_index("c")
    s = lax.axis_index("s")
    wid = s * NC + c
    pltpu.sync_copy(srcb.at[wid], isrc)
    pltpu.sync_copy(dstb.at[wid], idst)
    pltpu.sync_copy(ones_h, ones_v)
    pltpu.sync_copy(zz.at[pl.ds(s * RPS, RPS)], acc_o.at[pl.ds(s * RPS, RPS)])
    pltpu.sync_copy(zz.at[pl.ds(s * RPS, RPS)], acc_i.at[pl.ds(s * RPS, RPS)])
    plsc.subcore_barrier()

    def step(j, _):
        pltpu.sync_copy(ones_v, acc_o.at[isrc.at[j]], add=True)
        pltpu.sync_copy(ones_v, acc_i.at[idst.at[j]], add=True)
        return 0

    lax.fori_loop(0, NWIN, step, 0)
    plsc.subcore_barrier()
    pltpu.sync_copy(acc_o.at[pl.ds(s * RPS, RPS)], out.at[c, 0, pl.ds(s * RPS, RPS)])
    pltpu.sync_copy(acc_i.at[pl.ds(s * RPS, RPS)], out.at[c, 1, pl.ds(s * RPS, RPS)])


_degree_call = pl.kernel(
    _degree_body,
    out_type=jax.ShapeDtypeStruct((NC, 2, NP), jnp.float32),
    mesh=_MESH,
    compiler_params=pltpu.CompilerParams(use_tc_tiling_on_sc=False),
    scratch_types=[
        pltpu.VMEM((NWIN, WIN), jnp.int32),
        pltpu.VMEM((NWIN, WIN), jnp.int32),
        pltpu.VMEM((WIN,), jnp.float32),
        pltpu.VMEM_SHARED((NP,), jnp.float32),
        pltpu.VMEM_SHARED((NP,), jnp.float32),
    ],
)


def _prop_body(feat, srcb, dstb, zz, out,
               isrc, idst, rows0, rows1, acc, sg0, sg1, ss0, ss1):
    c = lax.axis_index("c")
    s = lax.axis_index("s")
    wid = s * NC + c
    pltpu.sync_copy(srcb.at[wid], isrc)
    pltpu.sync_copy(dstb.at[wid], idst)
    pltpu.sync_copy(zz.at[pl.ds(s * RPS, RPS)], acc.at[pl.ds(s * RPS, RPS)])
    plsc.subcore_barrier()

    def gather(j, buf, sem):
        pltpu.async_copy(feat.at[isrc.at[j]], buf, sem)

    def wait_gather(j, buf, sem):
        pltpu.make_async_copy(feat.at[isrc.at[j]], buf, sem).wait()

    def scatter(j, buf, sem):
        pltpu.async_copy(buf, acc.at[idst.at[j]], sem, add=True)

    def wait_scatter(j, buf, sem):
        pltpu.make_async_copy(buf, acc.at[idst.at[j]], sem).wait()

    gather(0, rows0, sg0)
    gather(1, rows1, sg1)

    def step(i, _):
        j0 = 2 * i
        j1 = 2 * i + 1
        wait_gather(j0, rows0, sg0)
        scatter(j0, rows0, ss0)
        wait_gather(j1, rows1, sg1)
        scatter(j1, rows1, ss1)
        wait_scatter(j0, rows0, ss0)

        @pl.when(j0 + 2 < NWIN)
        def _():
            gather(j0 + 2, rows0, sg0)

        wait_scatter(j1, rows1, ss1)

        @pl.when(j1 + 2 < NWIN)
        def _():
            gather(j1 + 2, rows1, sg1)

        return 0

    lax.fori_loop(0, NWIN // 2, step, 0)
    plsc.subcore_barrier()
    pltpu.sync_copy(acc.at[pl.ds(s * RPS, RPS)], out.at[c, pl.ds(s * RPS, RPS)])


_prop64 = pl.kernel(
    _prop_body,
    out_type=jax.ShapeDtypeStruct((NC, NP, FH), jnp.float32),
    mesh=_MESH,
    compiler_params=pltpu.CompilerParams(use_tc_tiling_on_sc=False),
    scratch_types=[
        pltpu.VMEM((NWIN, WIN), jnp.int32),
        pltpu.VMEM((NWIN, WIN), jnp.int32),
        pltpu.VMEM((WIN, FH), jnp.float32),
        pltpu.VMEM((WIN, FH), jnp.float32),
        pltpu.VMEM_SHARED((NP, FH), jnp.float32),
        pltpu.SemaphoreType.DMA,
        pltpu.SemaphoreType.DMA,
        pltpu.SemaphoreType.DMA,
        pltpu.SemaphoreType.DMA,
    ],
)


BR = 1024
_GRID = NP // BR


def _norms(deg_blk):
    do = deg_blk[:, 0] + deg_blk[:, 1]
    di = deg_blk[:, 2] + deg_blk[:, 3]
    ns = jnp.where(do > 0, lax.rsqrt(jnp.maximum(do, 1.0)), 0.0)
    nd = jnp.where(di > 0, lax.rsqrt(jnp.maximum(di, 1.0)), 0.0)
    return ns, nd


def _prep_body(x_ref, w_ref, deg_ref, ga_ref, gb_ref):
    ns, _ = _norms(deg_ref[...])
    t = jnp.dot(x_ref[...], w_ref[...], preferred_element_type=jnp.float32)
    g = t * ns[:, None]
    ga_ref[...] = g[:, :FH]
    gb_ref[...] = g[:, FH:]


def _mid_body(spa_ref, spb_ref, deg_ref, b_ref, w_ref, ga_ref, gb_ref):
    ns, nd = _norms(deg_ref[...])
    sfull = jnp.concatenate([spa_ref[0] + spa_ref[1], spb_ref[0] + spb_ref[1]],
                            axis=1)
    h = jnp.maximum(sfull * nd[:, None] + b_ref[0], 0.0)
    g = jnp.dot(h, w_ref[...], preferred_element_type=jnp.float32) * ns[:, None]
    ga_ref[...] = g[:, :FH]
    gb_ref[...] = g[:, FH:]


def _mid2_body(spa_ref, spb_ref, deg_ref, b_ref, w_ref, g_ref):
    ns, nd = _norms(deg_ref[...])
    sfull = jnp.concatenate([spa_ref[0] + spa_ref[1], spb_ref[0] + spb_ref[1]],
                            axis=1)
    h = jnp.maximum(sfull * nd[:, None] + b_ref[0], 0.0)
    g_ref[...] = jnp.dot(h, w_ref[...],
                         preferred_element_type=jnp.float32) * ns[:, None]


def _final_body(sp_ref, deg_ref, b_ref, o_ref):
    _, nd = _norms(deg_ref[...])
    o_ref[...] = (sp_ref[0] + sp_ref[1]) * nd[:, None] + b_ref[0]


def _row_spec(f):
    return pl.BlockSpec((BR, f), lambda i: (i, 0))


def _part_spec(f):
    return pl.BlockSpec((NC, BR, f), lambda i: (0, i, 0))


_DEG_SPEC = pl.BlockSpec((BR, 4), lambda i: (i, 0))


def _full_spec(shape):
    nd = len(shape)
    return pl.BlockSpec(shape, lambda i, _n=nd: (0,) * _n)


def _half_shapes():
    return (jax.ShapeDtypeStruct((NP, FH), jnp.float32),
            jax.ShapeDtypeStruct((NP, FH), jnp.float32))


def _tc_prep(x, w0, deg):
    return pl.pallas_call(
        _prep_body,
        grid=(_GRID,),
        in_specs=[_row_spec(F_IN), _full_spec((F_IN, F_HID)), _DEG_SPEC],
        out_specs=[_row_spec(FH), _row_spec(FH)],
        out_shape=_half_shapes(),
    )(x, w0, deg)


def _tc_mid(spa, spb, deg, b, w):
    return pl.pallas_call(
        _mid_body,
        grid=(_GRID,),
        in_specs=[_part_spec(FH), _part_spec(FH), _DEG_SPEC,
                  _full_spec((1, F_HID)), _full_spec((F_HID, F_HID))],
        out_specs=[_row_spec(FH), _row_spec(FH)],
        out_shape=_half_shapes(),
    )(spa, spb, deg, b, w)


def _tc_mid2(spa, spb, deg, b, w):
    return pl.pallas_call(
        _mid2_body,
        grid=(_GRID,),
        in_specs=[_part_spec(FH), _part_spec(FH), _DEG_SPEC,
                  _full_spec((1, F_HID)), _full_spec((F_HID, FH))],
        out_specs=_row_spec(FH),
        out_shape=jax.ShapeDtypeStruct((NP, FH), jnp.float32),
    )(spa, spb, deg, b, w)


def _tc_final(sp, deg, b):
    return pl.pallas_call(
        _final_body,
        grid=(_GRID,),
        in_specs=[_part_spec(FH), _DEG_SPEC, _full_spec((1, FH))],
        out_specs=_row_spec(FH),
        out_shape=jax.ShapeDtypeStruct((NP, FH), jnp.float32),
    )(sp, deg, b)


@jax.jit
def _pipeline(features, edge_index, W0, b0, W1, b1, W2, b2):
    src = edge_index[0].reshape(NWORK, NWIN, WIN)
    dst = edge_index[1].reshape(NWORK, NWIN, WIN)

    ones_w = jnp.ones((WIN,), jnp.float32)
    z_deg = jnp.zeros((NP,), jnp.float32)
    z64 = jnp.zeros((NP, FH), jnp.float32)
    xpad = jnp.zeros((NP, F_IN), jnp.float32).at[:N].set(features)

    dgo = _degree_call(src, dst, ones_w, z_deg)
    degp = jnp.stack([dgo[0, 0], dgo[1, 0], dgo[0, 1], dgo[1, 1]], axis=1)

    w2p = jnp.zeros((F_HID, FH), jnp.float32).at[:, :F_OUT].set(W2)
    b2p = jnp.zeros((1, FH), jnp.float32).at[0, :F_OUT].set(b2)

    g0a, g0b = _tc_prep(xpad, W0, degp)
    s0a = _prop64(g0a, src, dst, z64)
    s0b = _prop64(g0b, src, dst, z64)
    g1a, g1b = _tc_mid(s0a, s0b, degp, b0.reshape(1, -1), W1)
    s1a = _prop64(g1a, src, dst, z64)
    s1b = _prop64(g1b, src, dst, z64)
    g2 = _tc_mid2(s1a, s1b, degp, b1.reshape(1, -1), w2p)
    s2 = _prop64(g2, src, dst, z64)
    outp = _tc_final(s2, degp, b2p)
    return outp[:N, :F_OUT]


def kernel(features, edge_index, W0, b0, W1, b1, W2, b2):
    return _pipeline(features, edge_index, W0, b0, W1, b1, W2, b2)


# speedup vs baseline: 9.9333x; 1.1162x over previous
import jax
import jax.numpy as jnp
from jax import lax
from jax.experimental import pallas as pl
from jax.experimental.pallas import tpu as pltpu
from jax.experimental.pallas import tpu_sc as plsc

N = 10000
NP = 10240
E = 320000
F_IN = 128
F_HID = 128
F_OUT = 40

NC, NS = 2, 16
NWORK = NC * NS
EPW = E // NWORK
WIN = 100
NWIN = EPW // WIN
NBUF = 2
RPS = NP // NS

_MESH = plsc.VectorSubcoreMesh(core_axis_name="c", subcore_axis_name="s")
_SC_PARAMS = pltpu.CompilerParams(use_tc_tiling_on_sc=False)


def _degree_body(srcb, dstb, ones_h, zz, out, isrc, idst, ones_v, acc_o, acc_i):
    c = lax.axis_index("c")
    s = lax.axis_index("s")
    wid = s * NC + c
    pltpu.sync_copy(srcb.at[wid], isrc)
    pltpu.sync_copy(dstb.at[wid], idst)
    pltpu.sync_copy(ones_h, ones_v)
    pltpu.sync_copy(zz.at[pl.ds(s * RPS, RPS)], acc_o.at[pl.ds(s * RPS, RPS)])
    pltpu.sync_copy(zz.at[pl.ds(s * RPS, RPS)], acc_i.at[pl.ds(s * RPS, RPS)])
    plsc.subcore_barrier()

    def step(j, _):
        pltpu.sync_copy(ones_v, acc_o.at[isrc.at[j]], add=True)
        pltpu.sync_copy(ones_v, acc_i.at[idst.at[j]], add=True)
        return 0

    lax.fori_loop(0, NWIN, step, 0)
    plsc.subcore_barrier()
    pltpu.sync_copy(acc_o.at[pl.ds(s * RPS, RPS)], out.at[c, 0, pl.ds(s * RPS, RPS)])
    pltpu.sync_copy(acc_i.at[pl.ds(s * RPS, RPS)], out.at[c, 1, pl.ds(s * RPS, RPS)])


_degree_call = pl.kernel(
    _degree_body,
    out_type=jax.ShapeDtypeStruct((NC, 2, NP), jnp.float32),
    mesh=_MESH,
    compiler_params=_SC_PARAMS,
    scratch_types=[
        pltpu.VMEM((NWIN, WIN), jnp.int32),
        pltpu.VMEM((NWIN, WIN), jnp.int32),
        pltpu.VMEM((WIN,), jnp.float32),
        pltpu.VMEM_SHARED((NP,), jnp.float32),
        pltpu.VMEM_SHARED((NP,), jnp.float32),
    ],
)


def _prop_body(feat, srcb, dstb, zz, out,
               isrc, idst, rows, acc, sg, ss):
    c = lax.axis_index("c")
    s = lax.axis_index("s")
    wid = s * NC + c
    pltpu.sync_copy(srcb.at[wid], isrc)
    pltpu.sync_copy(dstb.at[wid], idst)
    pltpu.sync_copy(zz.at[pl.ds(s * RPS, RPS)], acc.at[pl.ds(s * RPS, RPS)])
    plsc.subcore_barrier()

    def gather(j, b):
        pltpu.async_copy(feat.at[isrc.at[j]], rows[b], sg[b])

    def wait_gather(j, b):
        pltpu.make_async_copy(feat.at[isrc.at[j]], rows[b], sg[b]).wait()

    def scatter(j, b):
        pltpu.async_copy(rows[b], acc.at[idst.at[j]], ss[b], add=True)

    def wait_scatter(j, b):
        pltpu.make_async_copy(rows[b], acc.at[idst.at[j]], ss[b]).wait()

    for b in range(NBUF):
        gather(b, b)

    def step(i, _):
        j = NBUF * i
        for b in range(NBUF):
            wait_gather(j + b, b)
            scatter(j + b, b)
        for b in range(NBUF):
            wait_scatter(j + b, b)

            @pl.when(j + b + NBUF < NWIN)
            def _():
                gather(j + b + NBUF, b)

        return 0

    lax.fori_loop(0, NWIN // NBUF, step, 0)
    plsc.subcore_barrier()
    pltpu.sync_copy(acc.at[pl.ds(s * RPS, RPS)], out.at[c, pl.ds(s * RPS, RPS)])


_prop128 = pl.kernel(
    _prop_body,
    out_type=jax.ShapeDtypeStruct((NC, NP, F_HID), jnp.float32),
    mesh=_MESH,
    compiler_params=_SC_PARAMS,
    scratch_types=[
        pltpu.VMEM((NWIN, WIN), jnp.int32),
        pltpu.VMEM((NWIN, WIN), jnp.int32),
        [pltpu.VMEM((WIN, F_HID), jnp.float32) for _ in range(NBUF)],
        pltpu.VMEM_SHARED((NP, F_HID), jnp.float32),
        [pltpu.SemaphoreType.DMA for _ in range(NBUF)],
        [pltpu.SemaphoreType.DMA for _ in range(NBUF)],
    ],
)


BR = 1024
_GRID = NP // BR


def _norms(deg_blk):
    do = deg_blk[0, 0] + deg_blk[1, 0]
    di = deg_blk[0, 1] + deg_blk[1, 1]
    ns = jnp.where(do > 0, lax.rsqrt(jnp.maximum(do, 1.0)), 0.0)
    nd = jnp.where(di > 0, lax.rsqrt(jnp.maximum(di, 1.0)), 0.0)
    return ns, nd


def _prep_body(x_ref, w_ref, deg_ref, g_ref):
    ns, _ = _norms(deg_ref[...])
    t = jnp.dot(x_ref[...], w_ref[...], preferred_element_type=jnp.float32)
    g_ref[...] = t * ns[:, None]


def _mid_body(sp_ref, deg_ref, b_ref, w_ref, g_ref):
    ns, nd = _norms(deg_ref[...])
    h = jnp.maximum((sp_ref[0] + sp_ref[1]) * nd[:, None] + b_ref[0], 0.0)
    g_ref[...] = jnp.dot(h, w_ref[...],
                         preferred_element_type=jnp.float32) * ns[:, None]


def _final_body(sp_ref, deg_ref, b_ref, o_ref):
    _, nd = _norms(deg_ref[...])
    o_ref[...] = (sp_ref[0] + sp_ref[1]) * nd[:, None] + b_ref[0]


def _row_spec(f):
    return pl.BlockSpec((BR, f), lambda i: (i, 0))


_PART_SPEC = pl.BlockSpec((NC, BR, F_HID), lambda i: (0, i, 0))
_DEG_SPEC = pl.BlockSpec((NC, 2, BR), lambda i: (0, 0, i))


def _full_spec(shape):
    nd = len(shape)
    return pl.BlockSpec(shape, lambda i, _n=nd: (0,) * _n)


def _tc_prep(x, w0, deg):
    return pl.pallas_call(
        _prep_body,
        grid=(_GRID,),
        in_specs=[_row_spec(F_IN), _full_spec((F_IN, F_HID)), _DEG_SPEC],
        out_specs=_row_spec(F_HID),
        out_shape=jax.ShapeDtypeStruct((NP, F_HID), jnp.float32),
    )(x, w0, deg)


def _tc_mid(sp, deg, b, w):
    return pl.pallas_call(
        _mid_body,
        grid=(_GRID,),
        in_specs=[_PART_SPEC, _DEG_SPEC,
                  _full_spec((1, F_HID)), _full_spec((F_HID, F_HID))],
        out_specs=_row_spec(F_HID),
        out_shape=jax.ShapeDtypeStruct((NP, F_HID), jnp.float32),
    )(sp, deg, b, w)


def _tc_final(sp, deg, b):
    return pl.pallas_call(
        _final_body,
        grid=(_GRID,),
        in_specs=[_PART_SPEC, _DEG_SPEC, _full_spec((1, F_HID))],
        out_specs=_row_spec(F_HID),
        out_shape=jax.ShapeDtypeStruct((NP, F_HID), jnp.float32),
    )(sp, deg, b)


@jax.jit
def _pipeline(features, edge_index, W0, b0, W1, b1, W2, b2):
    src = edge_index[0].reshape(NWORK, NWIN, WIN)
    dst = edge_index[1].reshape(NWORK, NWIN, WIN)

    ones_w = jnp.ones((WIN,), jnp.float32)
    z_deg = jnp.zeros((NP,), jnp.float32)
    z128 = jnp.zeros((NP, F_HID), jnp.float32)
    xpad = jnp.zeros((NP, F_IN), jnp.float32).at[:N].set(features)

    deg = _degree_call(src, dst, ones_w, z_deg)

    w2p = jnp.zeros((F_HID, F_HID), jnp.float32).at[:, :F_OUT].set(W2)
    b2p = jnp.zeros((1, F_HID), jnp.float32).at[0, :F_OUT].set(b2)

    g0 = _tc_prep(xpad, W0, deg)
    s0 = _prop128(g0, src, dst, z128)
    g1 = _tc_mid(s0, deg, b0.reshape(1, -1), W1)
    s1 = _prop128(g1, src, dst, z128)
    g2 = _tc_mid(s1, deg, b1.reshape(1, -1), w2p)
    s2 = _prop128(g2, src, dst, z128)
    outp = _tc_final(s2, deg, b2p)
    return outp[:N, :F_OUT]


def kernel(features, edge_index, W0, b0, W1, b1, W2, b2):
    return _pipeline(features, edge_index, W0, b0, W1, b1, W2, b2)


# speedup vs baseline: 12.2146x; 1.2297x over previous
import jax
import jax.numpy as jnp
from jax import lax
from jax.experimental import pallas as pl
from jax.experimental.pallas import tpu as pltpu
from jax.experimental.pallas import tpu_sc as plsc

N = 10000
NP = 10240
E = 320000
F_IN = 128
F_HID = 128
F_OUT = 40

NC, NS = 2, 16
NWORK = NC * NS
EPW = E // NWORK
WIN = 100
NWIN = EPW // WIN
FH = 64
GK = 5
NGRP = NWIN // GK
RPS = NP // NS

_MESH = plsc.VectorSubcoreMesh(core_axis_name="c", subcore_axis_name="s")
_SC_PARAMS = pltpu.CompilerParams(use_tc_tiling_on_sc=False)


def _degree_body(srcb, dstb, ones_h, zz, out, isrc, idst, ones_v, acc_o, acc_i):
    c = lax.axis_index("c")
    s = lax.axis_index("s")
    wid = s * NC + c
    pltpu.sync_copy(srcb.at[wid], isrc)
    pltpu.sync_copy(dstb.at[wid], idst)
    pltpu.sync_copy(ones_h, ones_v)
    pltpu.sync_copy(zz.at[pl.ds(s * RPS, RPS)], acc_o.at[pl.ds(s * RPS, RPS)])
    pltpu.sync_copy(zz.at[pl.ds(s * RPS, RPS)], acc_i.at[pl.ds(s * RPS, RPS)])
    plsc.subcore_barrier()

    def step(j, _):
        pltpu.sync_copy(ones_v, acc_o.at[isrc.at[j]], add=True)
        pltpu.sync_copy(ones_v, acc_i.at[idst.at[j]], add=True)
        return 0

    lax.fori_loop(0, NWIN, step, 0)
    plsc.subcore_barrier()
    pltpu.sync_copy(acc_o.at[pl.ds(s * RPS, RPS)], out.at[c, 0, pl.ds(s * RPS, RPS)])
    pltpu.sync_copy(acc_i.at[pl.ds(s * RPS, RPS)], out.at[c, 1, pl.ds(s * RPS, RPS)])


_degree_call = pl.kernel(
    _degree_body,
    out_type=jax.ShapeDtypeStruct((NC, 2, NP), jnp.float32),
    mesh=_MESH,
    compiler_params=_SC_PARAMS,
    scratch_types=[
        pltpu.VMEM((NWIN, WIN), jnp.int32),
        pltpu.VMEM((NWIN, WIN), jnp.int32),
        pltpu.VMEM((WIN,), jnp.float32),
        pltpu.VMEM_SHARED((NP,), jnp.float32),
        pltpu.VMEM_SHARED((NP,), jnp.float32),
    ],
)


def _prop_body(feat, srcb, dstb, zz, out,
               isrc, idst, rows, acc, sg, ss):
    c = lax.axis_index("c")
    s = lax.axis_index("s")
    wid = s * NC + c
    pltpu.sync_copy(srcb.at[wid], isrc)
    pltpu.sync_copy(dstb.at[wid], idst)
    pltpu.sync_copy(zz.at[pl.ds(s * RPS, RPS)], acc.at[pl.ds(s * RPS, RPS)])
    plsc.subcore_barrier()

    def _buf(st, b):
        return rows[st].at[pl.ds(b * WIN, WIN)]

    def issue_gathers(g, st):
        def one(b, _):
            pltpu.async_copy(feat.at[isrc.at[GK * g + b]], _buf(st, b), sg[st])
            return 0
        lax.fori_loop(0, GK, one, 0)

    def drain_gathers(g, st):
        def one(b, _):
            pltpu.make_async_copy(
                feat.at[isrc.at[GK * g + b]], _buf(st, b), sg[st]).wait()
            return 0
        lax.fori_loop(0, GK, one, 0)

    def issue_scatters(g, st):
        def one(b, _):
            pltpu.async_copy(
                _buf(st, b), acc.at[idst.at[GK * g + b]], ss[st], add=True)
            return 0
        lax.fori_loop(0, GK, one, 0)

    def drain_scatters(g, st):
        def one(b, _):
            pltpu.make_async_copy(
                _buf(st, b), acc.at[idst.at[GK * g + b]], ss[st]).wait()
            return 0
        lax.fori_loop(0, GK, one, 0)

    issue_gathers(0, 0)
    issue_gathers(1, 1)

    def half(g, st):
        drain_gathers(g, st)
        issue_scatters(g, st)
        drain_scatters(g, st)

        @pl.when(g + 2 < NGRP)
        def _():
            issue_gathers(g + 2, st)

    def step(i, _):
        half(2 * i, 0)
        half(2 * i + 1, 1)
        return 0

    lax.fori_loop(0, NGRP // 2, step, 0)
    if NGRP % 2:
        half(NGRP - 1, 0)
    plsc.subcore_barrier()
    pltpu.sync_copy(acc.at[pl.ds(s * RPS, RPS)], out.at[c, pl.ds(s * RPS, RPS)])


_prop64 = pl.kernel(
    _prop_body,
    out_type=jax.ShapeDtypeStruct((NC, NP, FH), jnp.float32),
    mesh=_MESH,
    compiler_params=_SC_PARAMS,
    scratch_types=[
        pltpu.VMEM((NWIN, WIN), jnp.int32),
        pltpu.VMEM((NWIN, WIN), jnp.int32),
        [pltpu.VMEM((GK * WIN, FH), jnp.float32) for _ in range(2)],
        pltpu.VMEM_SHARED((NP, FH), jnp.float32),
        [pltpu.SemaphoreType.DMA for _ in range(2)],
        [pltpu.SemaphoreType.DMA for _ in range(2)],
    ],
)


BR = 1024
_GRID = NP // BR


def _norms(deg_blk):
    do = deg_blk[0, 0] + deg_blk[1, 0]
    di = deg_blk[0, 1] + deg_blk[1, 1]
    ns = jnp.where(do > 0, lax.rsqrt(jnp.maximum(do, 1.0)), 0.0)
    nd = jnp.where(di > 0, lax.rsqrt(jnp.maximum(di, 1.0)), 0.0)
    return ns, nd


def _prep_body(x_ref, w_ref, deg_ref, ga_ref, gb_ref):
    ns, _ = _norms(deg_ref[...])
    t = jnp.dot(x_ref[...], w_ref[...], preferred_element_type=jnp.float32)
    g = t * ns[:, None]
    ga_ref[...] = g[:, :FH]
    gb_ref[...] = g[:, FH:]


def _mid_body(spa_ref, spb_ref, deg_ref, b_ref, w_ref, ga_ref, gb_ref):
    ns, nd = _norms(deg_ref[...])
    sfull = jnp.concatenate(
        [spa_ref[0] + spa_ref[1], spb_ref[0] + spb_ref[1]], axis=1)
    h = jnp.maximum(sfull * nd[:, None] + b_ref[0], 0.0)
    g = jnp.dot(h, w_ref[...], preferred_element_type=jnp.float32) * ns[:, None]
    ga_ref[...] = g[:, :FH]
    gb_ref[...] = g[:, FH:]


def _mid2_body(spa_ref, spb_ref, deg_ref, b_ref, w_ref, g_ref):
    ns, nd = _norms(deg_ref[...])
    sfull = jnp.concatenate(
        [spa_ref[0] + spa_ref[1], spb_ref[0] + spb_ref[1]], axis=1)
    h = jnp.maximum(sfull * nd[:, None] + b_ref[0], 0.0)
    g_ref[...] = jnp.dot(h, w_ref[...],
                         preferred_element_type=jnp.float32) * ns[:, None]


def _final_body(sp_ref, deg_ref, b_ref, o_ref):
    _, nd = _norms(deg_ref[...])
    o_ref[...] = (sp_ref[0] + sp_ref[1]) * nd[:, None] + b_ref[0]


def _row_spec(f):
    return pl.BlockSpec((BR, f), lambda i: (i, 0))


_PART_SPEC = pl.BlockSpec((NC, BR, FH), lambda i: (0, i, 0))
_DEG_SPEC = pl.BlockSpec((NC, 2, BR), lambda i: (0, 0, i))
_HALF_SHAPES = (jax.ShapeDtypeStruct((NP, FH), jnp.float32),
                jax.ShapeDtypeStruct((NP, FH), jnp.float32))


def _full_spec(shape):
    nd = len(shape)
    return pl.BlockSpec(shape, lambda i, _n=nd: (0,) * _n)


def _tc_prep(x, w0, deg):
    return pl.pallas_call(
        _prep_body,
        grid=(_GRID,),
        in_specs=[_row_spec(F_IN), _full_spec((F_IN, F_HID)), _DEG_SPEC],
        out_specs=[_row_spec(FH), _row_spec(FH)],
        out_shape=_HALF_SHAPES,
    )(x, w0, deg)


def _tc_mid(spa, spb, deg, b, w):
    return pl.pallas_call(
        _mid_body,
        grid=(_GRID,),
        in_specs=[_PART_SPEC, _PART_SPEC, _DEG_SPEC,
                  _full_spec((1, F_HID)), _full_spec((F_HID, F_HID))],
        out_specs=[_row_spec(FH), _row_spec(FH)],
        out_shape=_HALF_SHAPES,
    )(spa, spb, deg, b, w)


def _tc_mid2(spa, spb, deg, b, w):
    return pl.pallas_call(
        _mid2_body,
        grid=(_GRID,),
        in_specs=[_PART_SPEC, _PART_SPEC, _DEG_SPEC,
                  _full_spec((1, F_HID)), _full_spec((F_HID, FH))],
        out_specs=_row_spec(FH),
        out_shape=jax.ShapeDtypeStruct((NP, FH), jnp.float32),
    )(spa, spb, deg, b, w)


def _tc_final(sp, deg, b):
    return pl.pallas_call(
        _final_body,
        grid=(_GRID,),
        in_specs=[_PART_SPEC, _DEG_SPEC, _full_spec((1, FH))],
        out_specs=_row_spec(FH),
        out_shape=jax.ShapeDtypeStruct((NP, FH), jnp.float32),
    )(sp, deg, b)


@jax.jit
def _pipeline(features, edge_index, W0, b0, W1, b1, W2, b2):
    src = edge_index[0].reshape(NWORK, NWIN, WIN)
    dst = edge_index[1].reshape(NWORK, NWIN, WIN)

    ones_w = jnp.ones((WIN,), jnp.float32)
    z_deg = jnp.zeros((NP,), jnp.float32)
    z64 = jnp.zeros((NP, FH), jnp.float32)
    xpad = jnp.zeros((NP, F_IN), jnp.float32).at[:N].set(features)

    deg = _degree_call(src, dst, ones_w, z_deg)

    w2p = jnp.zeros((F_HID, FH), jnp.float32).at[:, :F_OUT].set(W2)
    b2p = jnp.zeros((1, FH), jnp.float32).at[0, :F_OUT].set(b2)

    g0a, g0b = _tc_prep(xpad, W0, deg)
    s0a = _prop64(g0a, src, dst, z64)
    s0b = _prop64(g0b, src, dst, z64)
    g1a, g1b = _tc_mid(s0a, s0b, deg, b0.reshape(1, -1), W1)
    s1a = _prop64(g1a, src, dst, z64)
    s1b = _prop64(g1b, src, dst, z64)
    g2 = _tc_mid2(s1a, s1b, deg, b1.reshape(1, -1), w2p)
    s2 = _prop64(g2, src, dst, z64)
    outp = _tc_final(s2, deg, b2p)
    return outp[:N, :F_OUT]


def kernel(features, edge_index, W0, b0, W1, b1, W2, b2):
    return _pipeline(features, edge_index, W0, b0, W1, b1, W2, b2)
